# baseline (speedup 1.0000x reference)
import functools

import jax
import jax.numpy as jnp
from jax import lax
from jax.experimental import pallas as pl
from jax.experimental.pallas import tpu as pltpu
from jax.experimental.pallas import tpu_sc as plsc

P = 6144
C = 64
NX = 512
NY = 512
NCELL = NX * NY
NB = 2
PB = P // NB
NRANGE = 16
RSIZE = NCELL // NRANGE
NCHUNK = PB // 16
WCAP = 384
OUTLEN = NB * C * NCELL

_mesh = plsc.VectorSubcoreMesh(core_axis_name="c", subcore_axis_name="s")


@functools.partial(
    pl.kernel,
    out_type=jax.ShapeDtypeStruct((NB, C, NY, NX), jnp.float32),
    mesh=_mesh,
    compiler_params=pltpu.CompilerParams(needs_layout_passes=False),
    scratch_types=[
        pltpu.VMEM((PB * 2,), jnp.int32),
        pltpu.VMEM((RSIZE,), jnp.int32),
        pltpu.VMEM((PB,), jnp.int32),
        pltpu.VMEM((PB,), jnp.int32),
        pltpu.VMEM((16,), jnp.int32),
        pltpu.VMEM((WCAP, 128), jnp.float32),
        pltpu.VMEM((RSIZE // NX, NX), jnp.float32),
        pltpu.VMEM((RSIZE // NX, NX), jnp.float32),
        pltpu.VMEM((RSIZE // NX, NX), jnp.float32),
        pltpu.SemaphoreType.DMA,
        pltpu.SemaphoreType.DMA,
        pltpu.SemaphoreType.DMA,
        pltpu.SemaphoreType.DMA,
    ],
)
def _pp_scatter(pf_hbm, vc_hbm, out_hbm, vc_v, table_v, finc_v, finp_v,
                s16_v, pair_v, slab0_v, slab1_v, slab2_v, gsem,
                s0sem, s1sem, s2sem):
    cid = lax.axis_index("c")
    sid = lax.axis_index("s")
    b = cid
    lo = sid * RSIZE
    iota = lax.iota(jnp.int32, 16)
    zvec = jnp.zeros((16,), jnp.float32)

    def zfill(i, _):
        o = i * 64
        for k in range(4):
            v = o + 16 * k + iota
            plsc.store_scatter(slab0_v, [v >> 9, v & (NX - 1)], zvec)
            plsc.store_scatter(slab1_v, [v >> 9, v & (NX - 1)], zvec)
            plsc.store_scatter(slab2_v, [v >> 9, v & (NX - 1)], zvec)
        return 0
    lax.fori_loop(0, RSIZE // 64, zfill, 0)

    pltpu.sync_copy(vc_hbm.at[pl.ds(b * (PB * 2), PB * 2)], vc_v)

    def my_cells(t):
        pvec = t * 16 + iota
        y = plsc.load_gather(vc_v, [pvec * 2])
        x = plsc.load_gather(vc_v, [pvec * 2 + 1])
        cell = y * NX + x
        valid = (cell >= lo) & (cell < lo + RSIZE)
        return pvec, cell, valid

    def phase1(t, count):
        pvec, cell, valid = my_cells(t)
        key = jnp.where(valid, cell * 8192 + pvec, jnp.int32(-1))
        skey = jnp.sort(key)
        s16_v[...] = skey
        nxt = plsc.load_gather(s16_v, [jnp.minimum(iota + 1, 15)])
        wcell = skey >> 13
        nxtc = jnp.where(iota == 15, jnp.int32(-2), nxt >> 13)
        winner = (skey >= 0) & (wcell != nxtc)
        tidx = jnp.where(winner, wcell - lo, 0)
        plsc.store_scatter(table_v, [tidx], skey & 8191, mask=winner)
        m32 = jnp.where(winner, jnp.int32(1), jnp.int32(0))
        dst = jnp.where(winner, count + jnp.cumsum(m32) - 1, 0)
        plsc.store_scatter(finc_v, [dst], wcell, mask=winner)
        plsc.store_scatter(finp_v, [dst], skey & 8191, mask=winner)
        return count + jnp.sum(m32)
    ncand = lax.fori_loop(0, NCHUNK, phase1, jnp.int32(0), unroll=2)

    def phase2(t, count):
        o = pl.multiple_of(t * 16, 16)
        cell = finc_v[pl.ds(o, 16)]
        pvec = finp_v[pl.ds(o, 16)]
        valid = (o + iota) < ncand
        tidx = jnp.where(valid, cell - lo, 0)
        w = plsc.load_gather(table_v, [tidx])
        final = valid & (w == pvec)
        m32 = jnp.where(final, jnp.int32(1), jnp.int32(0))
        dst = jnp.where(final, count + jnp.cumsum(m32) - 1, 0)
        plsc.store_scatter(finc_v, [dst], cell, mask=final)
        plsc.store_scatter(finp_v, [dst], pvec + b * PB, mask=final)
        return count + jnp.sum(m32)
    nwin = lax.fori_loop(0, (ncand + 15) // 16, phase2, jnp.int32(0))

    @pl.when(nwin > 0)
    def _pad():
        o = pl.multiple_of(((nwin - 1) // 16) * 16, 16)
        cv = finc_v[pl.ds(o, 16)]
        pv = finp_v[pl.ds(o, 16)]
        lasti = jnp.full((16,), nwin - 1, jnp.int32)
        lastc = plsc.load_gather(finc_v, [lasti])
        lastp = plsc.load_gather(finp_v, [lasti])
        inb = (o + iota) < nwin
        finc_v[pl.ds(o, 16)] = jnp.where(inb, cv, lastc)
        finp_v[pl.ds(o, 16)] = jnp.where(inb, pv, lastp)

    nwin16 = (nwin + 15) & ~15
    nch_fast = jnp.minimum(nwin16, WCAP) // 16

    def gfire(j, _):
        o = pl.multiple_of(j * 16, 16)
        pairidx = finp_v[pl.ds(o, 16)] >> 1
        pltpu.async_copy(pf_hbm.at[pairidx], pair_v.at[pl.ds(o, 16)], gsem)
        return 0
    lax.fori_loop(0, nch_fast, gfire, 0)

    def gdrain(j, _):
        pltpu.make_async_copy(
            pf_hbm.at[finp_v[pl.ds(0, 16)] >> 1],
            pair_v.at[pl.ds(0, 16)], gsem).wait()
        return 0
    lax.fori_loop(0, nch_fast, gdrain, 0)

    y0 = pl.multiple_of(sid * (RSIZE // NX), RSIZE // NX)

    def compose(slab, c, blk_base, nch):
        def body(j, _):
            o = pl.multiple_of(blk_base + j * 16, 16)
            cell = finc_v[pl.ds(o, 16)] - lo
            pid = finp_v[pl.ds(o, 16)]
            slot = j * 16 + iota
            col = (pid & 1) * 64 + c
            vals = plsc.load_gather(pair_v, [slot, col])
            plsc.store_scatter(slab, [cell >> 9, cell & (NX - 1)], vals)
            return 0
        lax.fori_loop(0, nch, body, 0)

    rings = ((slab0_v, s0sem), (slab1_v, s1sem), (slab2_v, s2sem))
    handles = []
    for c in range(C):
        slab, sem = rings[c % 3]
        if c >= 3:
            handles[c - 3].wait()
        compose(slab, c, 0, nch_fast)
        handles.append(pltpu.async_copy(
            slab, out_hbm.at[b, c, pl.ds(y0, RSIZE // NX), :], sem))
    for h in handles[C - 3:]:
        h.wait()

    @pl.when(nwin16 > WCAP)
    def _overflow():
        nblk = (nwin16 + WCAP - 1) // WCAP

        def do_block(k, _):
            base = k * WCAP
            nch = jnp.minimum(nwin16 - base, WCAP) // 16

            def gfire2(j, _):
                o = pl.multiple_of(base + j * 16, 16)
                ro = pl.multiple_of(j * 16, 16)
                pairidx = finp_v[pl.ds(o, 16)] >> 1
                pltpu.async_copy(
                    pf_hbm.at[pairidx], pair_v.at[pl.ds(ro, 16)], gsem)
                return 0
            lax.fori_loop(0, nch, gfire2, 0)
            lax.fori_loop(0, nch, gdrain, 0)

            def chan_rmw(c, _):
                dst = out_hbm.at[b, c, pl.ds(y0, RSIZE // NX), :]
                pltpu.sync_copy(dst, slab0_v)
                compose(slab0_v, c, base, nch)
                pltpu.sync_copy(slab0_v, dst)
                return 0
            lax.fori_loop(0, C, chan_rmw, 0)
            return 0
        lax.fori_loop(1, nblk, do_block, 0)


def kernel(pillar_features, voxel_coords):
    return _pp_scatter(
        pillar_features.reshape(PB, 2 * C),
        voxel_coords[:, 2:4].reshape(-1))


# speedup vs baseline: 1.0698x; 1.0698x over previous
import functools

import jax
import jax.numpy as jnp
from jax import lax
from jax.experimental import pallas as pl
from jax.experimental.pallas import tpu as pltpu
from jax.experimental.pallas import tpu_sc as plsc

P = 6144
C = 64
NX = 512
NY = 512
NCELL = NX * NY
NB = 2
PB = P // NB
NRANGE = 16
RSIZE = NCELL // NRANGE
NCHUNK = PB // 16
WCAP = 384
OUTLEN = NB * C * NCELL

_mesh = plsc.VectorSubcoreMesh(core_axis_name="c", subcore_axis_name="s")


@functools.partial(
    pl.kernel,
    out_type=jax.ShapeDtypeStruct((NB, C, NY, NX), jnp.float32),
    mesh=_mesh,
    compiler_params=pltpu.CompilerParams(needs_layout_passes=False),
    scratch_types=[
        pltpu.VMEM((PB * 2,), jnp.int32),
        pltpu.VMEM((RSIZE,), jnp.int32),
        pltpu.VMEM((PB,), jnp.int32),
        pltpu.VMEM((PB,), jnp.int32),
        pltpu.VMEM((16,), jnp.int32),
        pltpu.VMEM((WCAP, 128), jnp.float32),
        pltpu.VMEM((RSIZE // NX, NX), jnp.float32),
        pltpu.VMEM((RSIZE // NX, NX), jnp.float32),
        pltpu.SemaphoreType.DMA,
        pltpu.SemaphoreType.DMA,
        pltpu.SemaphoreType.DMA,
    ],
)
def _pp_scatter(pf_hbm, vc_hbm, out_hbm, vc_v, table_v, finc_v, finp_v,
                s16_v, pair_v, slab0_v, slab1_v, gsem, s0sem, s1sem):
    cid = lax.axis_index("c")
    sid = lax.axis_index("s")
    b = cid
    lo = sid * RSIZE
    iota = lax.iota(jnp.int32, 16)
    zvec = jnp.zeros((16,), jnp.float32)

    def zfill(i, _):
        o = i * 64
        for k in range(4):
            v = o + 16 * k + iota
            plsc.store_scatter(slab0_v, [v >> 9, v & (NX - 1)], zvec)
            plsc.store_scatter(slab1_v, [v >> 9, v & (NX - 1)], zvec)
        return 0
    lax.fori_loop(0, RSIZE // 64, zfill, 0)

    pltpu.sync_copy(vc_hbm.at[pl.ds(b * (PB * 2), PB * 2)], vc_v)

    def my_cells(t):
        pvec = t * 16 + iota
        y = plsc.load_gather(vc_v, [pvec * 2])
        x = plsc.load_gather(vc_v, [pvec * 2 + 1])
        cell = y * NX + x
        valid = (cell >= lo) & (cell < lo + RSIZE)
        return pvec, cell, valid

    def phase1(t, count):
        pvec, cell, valid = my_cells(t)
        key = jnp.where(valid, cell * 8192 + pvec, jnp.int32(-1))
        skey = jnp.sort(key)
        s16_v[...] = skey
        nxt = plsc.load_gather(s16_v, [jnp.minimum(iota + 1, 15)])
        wcell = skey >> 13
        nxtc = jnp.where(iota == 15, jnp.int32(-2), nxt >> 13)
        winner = (skey >= 0) & (wcell != nxtc)
        tidx = jnp.where(winner, wcell - lo, 0)
        plsc.store_scatter(table_v, [tidx], skey & 8191, mask=winner)
        m32 = jnp.where(winner, jnp.int32(1), jnp.int32(0))
        dst = jnp.where(winner, count + jnp.cumsum(m32) - 1, 0)
        plsc.store_scatter(finc_v, [dst], wcell, mask=winner)
        plsc.store_scatter(finp_v, [dst], skey & 8191, mask=winner)
        return count + jnp.sum(m32)
    ncand = lax.fori_loop(0, NCHUNK, phase1, jnp.int32(0))

    def phase2(t, count):
        o = pl.multiple_of(t * 16, 16)
        cell = finc_v[pl.ds(o, 16)]
        pvec = finp_v[pl.ds(o, 16)]
        valid = (o + iota) < ncand
        tidx = jnp.where(valid, cell - lo, 0)
        w = plsc.load_gather(table_v, [tidx])
        final = valid & (w == pvec)
        m32 = jnp.where(final, jnp.int32(1), jnp.int32(0))
        dst = jnp.where(final, count + jnp.cumsum(m32) - 1, 0)
        plsc.store_scatter(finc_v, [dst], cell, mask=final)
        plsc.store_scatter(finp_v, [dst], pvec + b * PB, mask=final)
        return count + jnp.sum(m32)
    nwin = lax.fori_loop(0, (ncand + 15) // 16, phase2, jnp.int32(0))

    @pl.when(nwin > 0)
    def _pad():
        o = pl.multiple_of(((nwin - 1) // 16) * 16, 16)
        cv = finc_v[pl.ds(o, 16)]
        pv = finp_v[pl.ds(o, 16)]
        lasti = jnp.full((16,), nwin - 1, jnp.int32)
        lastc = plsc.load_gather(finc_v, [lasti])
        lastp = plsc.load_gather(finp_v, [lasti])
        inb = (o + iota) < nwin
        finc_v[pl.ds(o, 16)] = jnp.where(inb, cv, lastc)
        finp_v[pl.ds(o, 16)] = jnp.where(inb, pv, lastp)

    nwin16 = (nwin + 15) & ~15
    nch_fast = jnp.minimum(nwin16, WCAP) // 16

    def gfire(j, _):
        o = pl.multiple_of(j * 16, 16)
        pairidx = finp_v[pl.ds(o, 16)] >> 1
        pltpu.async_copy(pf_hbm.at[pairidx], pair_v.at[pl.ds(o, 16)], gsem)
        return 0
    lax.fori_loop(0, nch_fast, gfire, 0)

    def gdrain(j, _):
        pltpu.make_async_copy(
            pf_hbm.at[finp_v[pl.ds(0, 16)] >> 1],
            pair_v.at[pl.ds(0, 16)], gsem).wait()
        return 0
    lax.fori_loop(0, nch_fast, gdrain, 0)

    y0 = pl.multiple_of(sid * (RSIZE // NX), RSIZE // NX)

    def compose(slab, c, blk_base, nch):
        def body(j, _):
            o = pl.multiple_of(blk_base + j * 16, 16)
            cell = finc_v[pl.ds(o, 16)] - lo
            pid = finp_v[pl.ds(o, 16)]
            slot = j * 16 + iota
            col = (pid & 1) * 64 + c
            vals = plsc.load_gather(pair_v, [slot, col])
            plsc.store_scatter(slab, [cell >> 9, cell & (NX - 1)], vals)
            return 0
        lax.fori_loop(0, nch, body, 0)

    def chan_pair(i, _):
        for par, slab, sem in ((0, slab0_v, s0sem), (1, slab1_v, s1sem)):
            c = i * 2 + par
            dst = out_hbm.at[b, c, pl.ds(y0, RSIZE // NX), :]

            @pl.when(i > 0)
            def _wait():
                pltpu.make_async_copy(slab, dst, sem).wait()

            compose(slab, c, 0, nch_fast)
            pltpu.async_copy(slab, dst, sem)
        return 0
    lax.fori_loop(0, C // 2, chan_pair, 0)

    for slab, sem, c in ((slab0_v, s0sem, C - 2), (slab1_v, s1sem, C - 1)):
        pltpu.make_async_copy(
            slab, out_hbm.at[b, c, pl.ds(y0, RSIZE // NX), :], sem).wait()

    @pl.when(nwin16 > WCAP)
    def _overflow():
        nblk = (nwin16 + WCAP - 1) // WCAP

        def do_block(k, _):
            base = k * WCAP
            nch = jnp.minimum(nwin16 - base, WCAP) // 16

            def gfire2(j, _):
                o = pl.multiple_of(base + j * 16, 16)
                ro = pl.multiple_of(j * 16, 16)
                pairidx = finp_v[pl.ds(o, 16)] >> 1
                pltpu.async_copy(
                    pf_hbm.at[pairidx], pair_v.at[pl.ds(ro, 16)], gsem)
                return 0
            lax.fori_loop(0, nch, gfire2, 0)
            lax.fori_loop(0, nch, gdrain, 0)

            def chan_rmw(c, _):
                dst = out_hbm.at[b, c, pl.ds(y0, RSIZE // NX), :]
                pltpu.sync_copy(dst, slab0_v)
                compose(slab0_v, c, base, nch)
                pltpu.sync_copy(slab0_v, dst)
                return 0
            lax.fori_loop(0, C, chan_rmw, 0)
            return 0
        lax.fori_loop(1, nblk, do_block, 0)


def kernel(pillar_features, voxel_coords):
    return _pp_scatter(
        pillar_features.reshape(PB, 2 * C),
        voxel_coords[:, 2:4].reshape(-1))


# speedup vs baseline: 1.0908x; 1.0196x over previous
import functools

import jax
import jax.numpy as jnp
from jax import lax
from jax.experimental import pallas as pl
from jax.experimental.pallas import tpu as pltpu
from jax.experimental.pallas import tpu_sc as plsc

P = 6144
C = 64
NX = 512
NY = 512
NCELL = NX * NY
NB = 2
PB = P // NB
NRANGE = 16
RSIZE = NCELL // NRANGE
NCHUNK = PB // 16
WCAP = 384
OUTLEN = NB * C * NCELL

_mesh = plsc.VectorSubcoreMesh(core_axis_name="c", subcore_axis_name="s")


@functools.partial(
    pl.kernel,
    out_type=jax.ShapeDtypeStruct((NB, C, NY, NX), jnp.float32),
    mesh=_mesh,
    compiler_params=pltpu.CompilerParams(needs_layout_passes=False),
    scratch_types=[
        pltpu.VMEM((PB * 2,), jnp.int32),
        pltpu.VMEM((RSIZE,), jnp.int32),
        pltpu.VMEM((PB,), jnp.int32),
        pltpu.VMEM((PB,), jnp.int32),
        pltpu.VMEM((16,), jnp.int32),
        pltpu.VMEM((WCAP, 128), jnp.float32),
        pltpu.VMEM((RSIZE // NX, NX), jnp.float32),
        pltpu.VMEM((RSIZE // NX, NX), jnp.float32),
        pltpu.SemaphoreType.DMA,
        pltpu.SemaphoreType.DMA,
        pltpu.SemaphoreType.DMA,
    ],
)
def _pp_scatter(pf_hbm, vc_hbm, out_hbm, vc_v, table_v, finc_v, finp_v,
                s16_v, pair_v, slab0_v, slab1_v, gsem, s0sem, s1sem):
    cid = lax.axis_index("c")
    sid = lax.axis_index("s")
    b = cid
    lo = sid * RSIZE
    iota = lax.iota(jnp.int32, 16)
    zvec = jnp.zeros((16,), jnp.float32)

    vc_copy = pltpu.async_copy(
        vc_hbm.at[pl.ds(b * (PB * 2), PB * 2)], vc_v, gsem)

    def zfill(i, _):
        o = i * 64
        for k in range(4):
            v = o + 16 * k + iota
            plsc.store_scatter(slab0_v, [v >> 9, v & (NX - 1)], zvec)
            plsc.store_scatter(slab1_v, [v >> 9, v & (NX - 1)], zvec)
        return 0
    lax.fori_loop(0, RSIZE // 64, zfill, 0)

    vc_copy.wait()

    def my_cells(t):
        pvec = t * 16 + iota
        y = plsc.load_gather(vc_v, [pvec * 2])
        x = plsc.load_gather(vc_v, [pvec * 2 + 1])
        cell = y * NX + x
        valid = (cell >= lo) & (cell < lo + RSIZE)
        return pvec, cell, valid

    def phase1(t, count):
        pvec, cell, valid = my_cells(t)
        key = jnp.where(valid, cell * 8192 + pvec, jnp.int32(-1))
        skey = jnp.sort(key)
        s16_v[...] = skey
        nxt = plsc.load_gather(s16_v, [jnp.minimum(iota + 1, 15)])
        wcell = skey >> 13
        nxtc = jnp.where(iota == 15, jnp.int32(-2), nxt >> 13)
        winner = (skey >= 0) & (wcell != nxtc)
        tidx = jnp.where(winner, wcell - lo, 0)
        plsc.store_scatter(table_v, [tidx], skey & 8191, mask=winner)
        m32 = jnp.where(winner, jnp.int32(1), jnp.int32(0))
        dst = jnp.where(winner, count + jnp.cumsum(m32) - 1, 0)
        plsc.store_scatter(finc_v, [dst], wcell, mask=winner)
        plsc.store_scatter(finp_v, [dst], skey & 8191, mask=winner)
        return count + jnp.sum(m32)
    ncand = lax.fori_loop(0, NCHUNK, phase1, jnp.int32(0))

    def phase2(t, count):
        o = pl.multiple_of(t * 16, 16)
        cell = finc_v[pl.ds(o, 16)]
        pvec = finp_v[pl.ds(o, 16)]
        valid = (o + iota) < ncand
        tidx = jnp.where(valid, cell - lo, 0)
        w = plsc.load_gather(table_v, [tidx])
        final = valid & (w == pvec)
        m32 = jnp.where(final, jnp.int32(1), jnp.int32(0))
        dst = jnp.where(final, count + jnp.cumsum(m32) - 1, 0)
        plsc.store_scatter(finc_v, [dst], cell, mask=final)
        plsc.store_scatter(finp_v, [dst], pvec + b * PB, mask=final)
        return count + jnp.sum(m32)
    nwin = lax.fori_loop(0, (ncand + 15) // 16, phase2, jnp.int32(0))

    @pl.when(nwin > 0)
    def _pad():
        o = pl.multiple_of(((nwin - 1) // 16) * 16, 16)
        cv = finc_v[pl.ds(o, 16)]
        pv = finp_v[pl.ds(o, 16)]
        lasti = jnp.full((16,), nwin - 1, jnp.int32)
        lastc = plsc.load_gather(finc_v, [lasti])
        lastp = plsc.load_gather(finp_v, [lasti])
        inb = (o + iota) < nwin
        finc_v[pl.ds(o, 16)] = jnp.where(inb, cv, lastc)
        finp_v[pl.ds(o, 16)] = jnp.where(inb, pv, lastp)

    nwin16 = (nwin + 15) & ~15
    nch_fast = jnp.minimum(nwin16, WCAP) // 16

    def gfire(j, _):
        o = pl.multiple_of(j * 16, 16)
        pairidx = finp_v[pl.ds(o, 16)] >> 1
        pltpu.async_copy(pf_hbm.at[pairidx], pair_v.at[pl.ds(o, 16)], gsem)
        return 0
    lax.fori_loop(0, nch_fast, gfire, 0)

    def gdrain(j, _):
        pltpu.make_async_copy(
            pf_hbm.at[finp_v[pl.ds(0, 16)] >> 1],
            pair_v.at[pl.ds(0, 16)], gsem).wait()
        return 0
    lax.fori_loop(0, nch_fast, gdrain, 0)

    y0 = pl.multiple_of(sid * (RSIZE // NX), RSIZE // NX)

    def compose(slab, c, blk_base, nch):
        def body(j, _):
            o = pl.multiple_of(blk_base + j * 16, 16)
            cell = finc_v[pl.ds(o, 16)] - lo
            pid = finp_v[pl.ds(o, 16)]
            slot = j * 16 + iota
            col = (pid & 1) * 64 + c
            vals = plsc.load_gather(pair_v, [slot, col])
            plsc.store_scatter(slab, [cell >> 9, cell & (NX - 1)], vals)
            return 0
        lax.fori_loop(0, nch, body, 0)

    def chan_pair(i, _):
        for par, slab, sem in ((0, slab0_v, s0sem), (1, slab1_v, s1sem)):
            c = i * 2 + par
            dst = out_hbm.at[b, c, pl.ds(y0, RSIZE // NX), :]

            @pl.when(i > 0)
            def _wait():
                pltpu.make_async_copy(slab, dst, sem).wait()

            compose(slab, c, 0, nch_fast)
            pltpu.async_copy(slab, dst, sem)
        return 0
    lax.fori_loop(0, C // 2, chan_pair, 0)

    for slab, sem, c in ((slab0_v, s0sem, C - 2), (slab1_v, s1sem, C - 1)):
        pltpu.make_async_copy(
            slab, out_hbm.at[b, c, pl.ds(y0, RSIZE // NX), :], sem).wait()

    @pl.when(nwin16 > WCAP)
    def _overflow():
        nblk = (nwin16 + WCAP - 1) // WCAP

        def do_block(k, _):
            base = k * WCAP
            nch = jnp.minimum(nwin16 - base, WCAP) // 16

            def gfire2(j, _):
                o = pl.multiple_of(base + j * 16, 16)
                ro = pl.multiple_of(j * 16, 16)
                pairidx = finp_v[pl.ds(o, 16)] >> 1
                pltpu.async_copy(
                    pf_hbm.at[pairidx], pair_v.at[pl.ds(ro, 16)], gsem)
                return 0
            lax.fori_loop(0, nch, gfire2, 0)
            lax.fori_loop(0, nch, gdrain, 0)

            def chan_rmw(c, _):
                dst = out_hbm.at[b, c, pl.ds(y0, RSIZE // NX), :]
                pltpu.sync_copy(dst, slab0_v)
                compose(slab0_v, c, base, nch)
                pltpu.sync_copy(slab0_v, dst)
                return 0
            lax.fori_loop(0, C, chan_rmw, 0)
            return 0
        lax.fori_loop(1, nblk, do_block, 0)


def kernel(pillar_features, voxel_coords):
    return _pp_scatter(
        pillar_features.reshape(PB, 2 * C),
        voxel_coords[:, 2:4].reshape(-1))


# speedup vs baseline: 1.1191x; 1.0259x over previous
import functools

import jax
import jax.numpy as jnp
from jax import lax
from jax.experimental import pallas as pl
from jax.experimental.pallas import tpu as pltpu
from jax.experimental.pallas import tpu_sc as plsc

P = 6144
C = 64
NX = 512
NY = 512
NCELL = NX * NY
NB = 2
PB = P // NB
NRANGE = 16
RSIZE = NCELL // NRANGE
NCHUNK = PB // 16
WCAP = 384
OUTLEN = NB * C * NCELL

_mesh = plsc.VectorSubcoreMesh(core_axis_name="c", subcore_axis_name="s")


@functools.partial(
    pl.kernel,
    out_type=jax.ShapeDtypeStruct((NB, C, NY, NX), jnp.float32),
    mesh=_mesh,
    compiler_params=pltpu.CompilerParams(needs_layout_passes=False),
    scratch_types=[
        pltpu.VMEM((PB,), jnp.int32),
        pltpu.VMEM((PB,), jnp.int32),
        pltpu.VMEM((RSIZE,), jnp.int32),
        pltpu.VMEM((PB,), jnp.int32),
        pltpu.VMEM((PB,), jnp.int32),
        pltpu.VMEM((16,), jnp.int32),
        pltpu.VMEM((WCAP, 128), jnp.float32),
        pltpu.VMEM((RSIZE // NX, NX), jnp.float32),
        pltpu.VMEM((RSIZE // NX, NX), jnp.float32),
        pltpu.SemaphoreType.DMA,
        pltpu.SemaphoreType.DMA,
        pltpu.SemaphoreType.DMA,
    ],
)
def _pp_scatter(pf_hbm, ys_hbm, xs_hbm, out_hbm, yv_v, xv_v, table_v,
                finc_v, finp_v, s16_v, pair_v, slab0_v, slab1_v,
                gsem, s0sem, s1sem):
    cid = lax.axis_index("c")
    sid = lax.axis_index("s")
    b = cid
    lo = sid * RSIZE
    iota = lax.iota(jnp.int32, 16)
    zvec = jnp.zeros((16,), jnp.float32)

    y_copy = pltpu.async_copy(ys_hbm.at[pl.ds(b * PB, PB)], yv_v, gsem)
    x_copy = pltpu.async_copy(xs_hbm.at[pl.ds(b * PB, PB)], xv_v, gsem)

    def zfill(i, _):
        o = i * 64
        for k in range(4):
            v = o + 16 * k + iota
            plsc.store_scatter(slab0_v, [v >> 9, v & (NX - 1)], zvec)
            plsc.store_scatter(slab1_v, [v >> 9, v & (NX - 1)], zvec)
        return 0
    lax.fori_loop(0, RSIZE // 64, zfill, 0)

    y_copy.wait()
    x_copy.wait()

    def my_cells(t):
        pvec = t * 16 + iota
        o = pl.multiple_of(t * 16, 16)
        cell = yv_v[pl.ds(o, 16)] * NX + xv_v[pl.ds(o, 16)]
        valid = (cell >= lo) & (cell < lo + RSIZE)
        return pvec, cell, valid

    def phase1(t, count):
        pvec, cell, valid = my_cells(t)
        key = jnp.where(valid, cell * 8192 + pvec, jnp.int32(-1))
        skey = jnp.sort(key)
        s16_v[...] = skey
        nxt = plsc.load_gather(s16_v, [jnp.minimum(iota + 1, 15)])
        wcell = skey >> 13
        nxtc = jnp.where(iota == 15, jnp.int32(-2), nxt >> 13)
        winner = (skey >= 0) & (wcell != nxtc)
        tidx = jnp.where(winner, wcell - lo, 0)
        plsc.store_scatter(table_v, [tidx], skey & 8191, mask=winner)
        m32 = jnp.where(winner, jnp.int32(1), jnp.int32(0))
        dst = jnp.where(winner, count + jnp.cumsum(m32) - 1, 0)
        plsc.store_scatter(finc_v, [dst], wcell, mask=winner)
        plsc.store_scatter(finp_v, [dst], skey & 8191, mask=winner)
        return count + jnp.sum(m32)
    ncand = lax.fori_loop(0, NCHUNK, phase1, jnp.int32(0))

    def phase2(t, count):
        o = pl.multiple_of(t * 16, 16)
        cell = finc_v[pl.ds(o, 16)]
        pvec = finp_v[pl.ds(o, 16)]
        valid = (o + iota) < ncand
        tidx = jnp.where(valid, cell - lo, 0)
        w = plsc.load_gather(table_v, [tidx])
        final = valid & (w == pvec)
        m32 = jnp.where(final, jnp.int32(1), jnp.int32(0))
        dst = jnp.where(final, count + jnp.cumsum(m32) - 1, 0)
        plsc.store_scatter(finc_v, [dst], cell, mask=final)
        plsc.store_scatter(finp_v, [dst], pvec + b * PB, mask=final)
        return count + jnp.sum(m32)
    nwin = lax.fori_loop(0, (ncand + 15) // 16, phase2, jnp.int32(0))

    @pl.when(nwin > 0)
    def _pad():
        o = pl.multiple_of(((nwin - 1) // 16) * 16, 16)
        cv = finc_v[pl.ds(o, 16)]
        pv = finp_v[pl.ds(o, 16)]
        lasti = jnp.full((16,), nwin - 1, jnp.int32)
        lastc = plsc.load_gather(finc_v, [lasti])
        lastp = plsc.load_gather(finp_v, [lasti])
        inb = (o + iota) < nwin
        finc_v[pl.ds(o, 16)] = jnp.where(inb, cv, lastc)
        finp_v[pl.ds(o, 16)] = jnp.where(inb, pv, lastp)

    nwin16 = (nwin + 15) & ~15
    nch_fast = jnp.minimum(nwin16, WCAP) // 16

    def gfire(j, _):
        o = pl.multiple_of(j * 16, 16)
        pairidx = finp_v[pl.ds(o, 16)] >> 1
        pltpu.async_copy(pf_hbm.at[pairidx], pair_v.at[pl.ds(o, 16)], gsem)
        return 0
    lax.fori_loop(0, nch_fast, gfire, 0)

    def gdrain(j, _):
        pltpu.make_async_copy(
            pf_hbm.at[finp_v[pl.ds(0, 16)] >> 1],
            pair_v.at[pl.ds(0, 16)], gsem).wait()
        return 0
    lax.fori_loop(0, nch_fast, gdrain, 0)

    y0 = pl.multiple_of(sid * (RSIZE // NX), RSIZE // NX)

    def compose(slab, c, blk_base, nch):
        def body(j, _):
            o = pl.multiple_of(blk_base + j * 16, 16)
            cell = finc_v[pl.ds(o, 16)] - lo
            pid = finp_v[pl.ds(o, 16)]
            slot = j * 16 + iota
            col = (pid & 1) * 64 + c
            vals = plsc.load_gather(pair_v, [slot, col])
            plsc.store_scatter(slab, [cell >> 9, cell & (NX - 1)], vals)
            return 0
        lax.fori_loop(0, nch, body, 0)

    def chan_pair(i, _):
        for par, slab, sem in ((0, slab0_v, s0sem), (1, slab1_v, s1sem)):
            c = i * 2 + par
            dst = out_hbm.at[b, c, pl.ds(y0, RSIZE // NX), :]

            @pl.when(i > 0)
            def _wait():
                pltpu.make_async_copy(slab, dst, sem).wait()

            compose(slab, c, 0, nch_fast)
            pltpu.async_copy(slab, dst, sem)
        return 0
    lax.fori_loop(0, C // 2, chan_pair, 0)

    for slab, sem, c in ((slab0_v, s0sem, C - 2), (slab1_v, s1sem, C - 1)):
        pltpu.make_async_copy(
            slab, out_hbm.at[b, c, pl.ds(y0, RSIZE // NX), :], sem).wait()

    @pl.when(nwin16 > WCAP)
    def _overflow():
        nblk = (nwin16 + WCAP - 1) // WCAP

        def do_block(k, _):
            base = k * WCAP
            nch = jnp.minimum(nwin16 - base, WCAP) // 16

            def gfire2(j, _):
                o = pl.multiple_of(base + j * 16, 16)
                ro = pl.multiple_of(j * 16, 16)
                pairidx = finp_v[pl.ds(o, 16)] >> 1
                pltpu.async_copy(
                    pf_hbm.at[pairidx], pair_v.at[pl.ds(ro, 16)], gsem)
                return 0
            lax.fori_loop(0, nch, gfire2, 0)
            lax.fori_loop(0, nch, gdrain, 0)

            def chan_rmw(c, _):
                dst = out_hbm.at[b, c, pl.ds(y0, RSIZE // NX), :]
                pltpu.sync_copy(dst, slab0_v)
                compose(slab0_v, c, base, nch)
                pltpu.sync_copy(slab0_v, dst)
                return 0
            lax.fori_loop(0, C, chan_rmw, 0)
            return 0
        lax.fori_loop(1, nblk, do_block, 0)


def kernel(pillar_features, voxel_coords):
    return _pp_scatter(
        pillar_features.reshape(PB, 2 * C),
        voxel_coords[:, 2], voxel_coords[:, 3])


# speedup vs baseline: 1.1199x; 1.0008x over previous
import functools

import jax
import jax.numpy as jnp
from jax import lax
from jax.experimental import pallas as pl
from jax.experimental.pallas import tpu as pltpu
from jax.experimental.pallas import tpu_sc as plsc

P = 6144
C = 64
NX = 512
NY = 512
NCELL = NX * NY
NB = 2
PB = P // NB
NRANGE = 16
RSIZE = NCELL // NRANGE
NCHUNK = PB // 16
WCAP = 384
OUTLEN = NB * C * NCELL

_mesh = plsc.VectorSubcoreMesh(core_axis_name="c", subcore_axis_name="s")


@functools.partial(
    pl.kernel,
    out_type=jax.ShapeDtypeStruct((NB, C, NY, NX), jnp.float32),
    mesh=_mesh,
    compiler_params=pltpu.CompilerParams(needs_layout_passes=False),
    scratch_types=[
        pltpu.VMEM((PB,), jnp.int32),
        pltpu.VMEM((PB,), jnp.int32),
        pltpu.VMEM((RSIZE,), jnp.int32),
        pltpu.VMEM((PB,), jnp.int32),
        pltpu.VMEM((PB,), jnp.int32),
        pltpu.VMEM((16,), jnp.int32),
        pltpu.VMEM((WCAP, 128), jnp.float32),
        pltpu.VMEM((RSIZE // NX, NX), jnp.float32),
        pltpu.VMEM((RSIZE // NX, NX), jnp.float32),
        pltpu.SemaphoreType.DMA,
        pltpu.SemaphoreType.DMA,
        pltpu.SemaphoreType.DMA,
    ],
)
def _pp_scatter(pf_hbm, yx_hbm, out_hbm, yv_v, xv_v, table_v,
                finc_v, finp_v, s16_v, pair_v, slab0_v, slab1_v,
                gsem, s0sem, s1sem):
    cid = lax.axis_index("c")
    sid = lax.axis_index("s")
    b = cid
    lo = sid * RSIZE
    iota = lax.iota(jnp.int32, 16)
    zvec = jnp.zeros((16,), jnp.float32)

    y_copy = pltpu.async_copy(yx_hbm.at[pl.ds(b * PB, PB)], yv_v, gsem)
    x_copy = pltpu.async_copy(yx_hbm.at[pl.ds(P + b * PB, PB)], xv_v, gsem)

    def zfill(i, _):
        o = i * 64
        for k in range(4):
            v = o + 16 * k + iota
            plsc.store_scatter(slab0_v, [v >> 9, v & (NX - 1)], zvec)
            plsc.store_scatter(slab1_v, [v >> 9, v & (NX - 1)], zvec)
        return 0
    lax.fori_loop(0, RSIZE // 64, zfill, 0)

    y_copy.wait()
    x_copy.wait()

    def my_cells(t):
        pvec = t * 16 + iota
        o = pl.multiple_of(t * 16, 16)
        cell = yv_v[pl.ds(o, 16)] * NX + xv_v[pl.ds(o, 16)]
        valid = (cell >= lo) & (cell < lo + RSIZE)
        return pvec, cell, valid

    def phase1(t, count):
        pvec, cell, valid = my_cells(t)
        key = jnp.where(valid, cell * 8192 + pvec, jnp.int32(-1))
        skey = jnp.sort(key)
        s16_v[...] = skey
        nxt = plsc.load_gather(s16_v, [jnp.minimum(iota + 1, 15)])
        wcell = skey >> 13
        nxtc = jnp.where(iota == 15, jnp.int32(-2), nxt >> 13)
        winner = (skey >= 0) & (wcell != nxtc)
        tidx = jnp.where(winner, wcell - lo, 0)
        plsc.store_scatter(table_v, [tidx], skey & 8191, mask=winner)
        m32 = jnp.where(winner, jnp.int32(1), jnp.int32(0))
        dst = jnp.where(winner, count + jnp.cumsum(m32) - 1, 0)
        plsc.store_scatter(finc_v, [dst], wcell, mask=winner)
        plsc.store_scatter(finp_v, [dst], skey & 8191, mask=winner)
        return count + jnp.sum(m32)
    ncand = lax.fori_loop(0, NCHUNK, phase1, jnp.int32(0))

    def phase2(t, count):
        o = pl.multiple_of(t * 16, 16)
        cell = finc_v[pl.ds(o, 16)]
        pvec = finp_v[pl.ds(o, 16)]
        valid = (o + iota) < ncand
        tidx = jnp.where(valid, cell - lo, 0)
        w = plsc.load_gather(table_v, [tidx])
        final = valid & (w == pvec)
        m32 = jnp.where(final, jnp.int32(1), jnp.int32(0))
        dst = jnp.where(final, count + jnp.cumsum(m32) - 1, 0)
        plsc.store_scatter(finc_v, [dst], cell, mask=final)
        plsc.store_scatter(finp_v, [dst], pvec + b * PB, mask=final)
        return count + jnp.sum(m32)
    nwin = lax.fori_loop(0, (ncand + 15) // 16, phase2, jnp.int32(0))

    @pl.when(nwin > 0)
    def _pad():
        o = pl.multiple_of(((nwin - 1) // 16) * 16, 16)
        cv = finc_v[pl.ds(o, 16)]
        pv = finp_v[pl.ds(o, 16)]
        lasti = jnp.full((16,), nwin - 1, jnp.int32)
        lastc = plsc.load_gather(finc_v, [lasti])
        lastp = plsc.load_gather(finp_v, [lasti])
        inb = (o + iota) < nwin
        finc_v[pl.ds(o, 16)] = jnp.where(inb, cv, lastc)
        finp_v[pl.ds(o, 16)] = jnp.where(inb, pv, lastp)

    nwin16 = (nwin + 15) & ~15
    nch_fast = jnp.minimum(nwin16, WCAP) // 16

    def gfire(j, _):
        o = pl.multiple_of(j * 16, 16)
        pairidx = finp_v[pl.ds(o, 16)] >> 1
        pltpu.async_copy(pf_hbm.at[pairidx], pair_v.at[pl.ds(o, 16)], gsem)
        return 0
    lax.fori_loop(0, nch_fast, gfire, 0)

    def gdrain(j, _):
        pltpu.make_async_copy(
            pf_hbm.at[finp_v[pl.ds(0, 16)] >> 1],
            pair_v.at[pl.ds(0, 16)], gsem).wait()
        return 0
    lax.fori_loop(0, nch_fast, gdrain, 0)

    y0 = pl.multiple_of(sid * (RSIZE // NX), RSIZE // NX)

    def compose(slab, c, blk_base, nch):
        def body(j, _):
            o = pl.multiple_of(blk_base + j * 16, 16)
            cell = finc_v[pl.ds(o, 16)] - lo
            pid = finp_v[pl.ds(o, 16)]
            slot = j * 16 + iota
            col = (pid & 1) * 64 + c
            vals = plsc.load_gather(pair_v, [slot, col])
            plsc.store_scatter(slab, [cell >> 9, cell & (NX - 1)], vals)
            return 0
        lax.fori_loop(0, nch, body, 0)

    def chan_pair(i, _):
        for par, slab, sem in ((0, slab0_v, s0sem), (1, slab1_v, s1sem)):
            c = i * 2 + par
            dst = out_hbm.at[b, c, pl.ds(y0, RSIZE // NX), :]

            @pl.when(i > 0)
            def _wait():
                pltpu.make_async_copy(slab, dst, sem).wait()

            compose(slab, c, 0, nch_fast)
            pltpu.async_copy(slab, dst, sem)
        return 0
    lax.fori_loop(0, C // 2, chan_pair, 0)

    for slab, sem, c in ((slab0_v, s0sem, C - 2), (slab1_v, s1sem, C - 1)):
        pltpu.make_async_copy(
            slab, out_hbm.at[b, c, pl.ds(y0, RSIZE // NX), :], sem).wait()

    @pl.when(nwin16 > WCAP)
    def _overflow():
        nblk = (nwin16 + WCAP - 1) // WCAP

        def do_block(k, _):
            base = k * WCAP
            nch = jnp.minimum(nwin16 - base, WCAP) // 16

            def gfire2(j, _):
                o = pl.multiple_of(base + j * 16, 16)
                ro = pl.multiple_of(j * 16, 16)
                pairidx = finp_v[pl.ds(o, 16)] >> 1
                pltpu.async_copy(
                    pf_hbm.at[pairidx], pair_v.at[pl.ds(ro, 16)], gsem)
                return 0
            lax.fori_loop(0, nch, gfire2, 0)
            lax.fori_loop(0, nch, gdrain, 0)

            def chan_rmw(c, _):
                dst = out_hbm.at[b, c, pl.ds(y0, RSIZE // NX), :]
                pltpu.sync_copy(dst, slab0_v)
                compose(slab0_v, c, base, nch)
                pltpu.sync_copy(slab0_v, dst)
                return 0
            lax.fori_loop(0, C, chan_rmw, 0)
            return 0
        lax.fori_loop(1, nblk, do_block, 0)


def kernel(pillar_features, voxel_coords):
    return _pp_scatter(
        pillar_features.reshape(PB, 2 * C),
        voxel_coords[:, 2:4].T.reshape(-1))
